# Initial kernel scaffold; baseline (speedup 1.0000x reference)
import jax
import jax.numpy as jnp
from jax.experimental import pallas as pl


def kernel(champions, items, traits, scalars, champ_table, item_table, trait_table, bench_table, pid_table, mlp_w1, mlp_b1, mlp_w2, mlp_b2):
    raise NotImplementedError("write your pallas kernel here")


# speedup vs baseline: 10.4451x; 10.4451x over previous
import jax
import jax.numpy as jnp
from jax.experimental import pallas as pl

VEC_D = 135
N_CH = 37
N_BENCH = 10
N_SCAL = 6
N_PID = 4
STATS_D = 19
XDIM = 60 + 3 * 60 + 7 * 27 + STATS_D


def _onehot_f(col, n):
    idx = jnp.clip(jnp.floor(col), 0.0, float(n - 1)).astype(jnp.int32)
    i = jax.lax.broadcasted_iota(jnp.int32, (col.shape[0], n), 1)
    return (i == idx).astype(jnp.float32)


def _emb_kernel(ch_ref, it_ref, tr_ref, sc_ref, t_ref, bench_ref, pid_ref,
                w1_ref, b1_ref, w2_ref, b2_ref, out_ref):
    bb = ch_ref.shape[0]

    ch3 = ch_ref[...]
    CH = jnp.concatenate([ch3[:, c, :] for c in range(N_CH)], axis=0)
    pieces = [_onehot_f(CH[:, 0:1], 60)]
    for k in range(3):
        pieces.append(_onehot_f(CH[:, 1 + k:2 + k], 60))
    for j in range(7):
        pieces.append(_onehot_f(CH[:, 4 + j:5 + j], 27))
    pieces.append(CH[:, 11:30])
    X = jnp.concatenate(pieces, axis=1)
    rows = jax.lax.dot_general(X, t_ref[...], (((1,), (0,)), ((), ())),
                               preferred_element_type=jnp.float32)
    for c in range(N_CH):
        out_ref[:, c, :] = rows[c * bb:(c + 1) * bb, :]

    it = it_ref[...]
    IT = jnp.concatenate([it[:, k:k + 1] for k in range(N_BENCH)], axis=0)
    bench_rows = jax.lax.dot_general(_onehot_f(IT, 60), bench_ref[...],
                                     (((1,), (0,)), ((), ())),
                                     preferred_element_type=jnp.float32)
    for k in range(N_BENCH):
        out_ref[:, N_CH + k, :] = bench_rows[k * bb:(k + 1) * bb, :]

    h = jax.lax.dot_general(tr_ref[...], w1_ref[...], (((1,), (0,)), ((), ())),
                            preferred_element_type=jnp.float32) + b1_ref[...]
    h = jnp.maximum(h, 0.0)
    out_ref[:, 47, :] = jax.lax.dot_general(h, w2_ref[...],
                                            (((1,), (0,)), ((), ())),
                                            preferred_element_type=jnp.float32) + b2_ref[...]

    sc = sc_ref[...]
    step = 200.0 / (VEC_D - 1)
    rng = jax.lax.broadcasted_iota(jnp.int32, (bb, VEC_D), 1)
    for k in range(N_SCAL):
        v = jnp.clip(sc[:, 1 + k:2 + k], 0.0, 200.0)
        steps = v / step
        lower = jnp.floor(steps)
        um = steps - lower
        lower_i = lower.astype(jnp.int32)
        enc = (rng == lower_i).astype(jnp.float32) * (1.0 - um) + \
              (rng == lower_i + 1).astype(jnp.float32) * um
        out_ref[:, 48 + k, :] = enc

    PID = jnp.concatenate([sc[:, 7:8], sc[:, 8:9], sc[:, 9:10], sc[:, 0:1]],
                          axis=0)
    pid_rows = jax.lax.dot_general(_onehot_f(PID, 8), pid_ref[...],
                                   (((1,), (0,)), ((), ())),
                                   preferred_element_type=jnp.float32)
    for k in range(N_PID):
        out_ref[:, N_CH + N_BENCH + 1 + N_SCAL + k, :] = \
            pid_rows[k * bb:(k + 1) * bb, :]


def kernel(champions, items, traits, scalars, champ_table, item_table,
           trait_table, bench_table, pid_table, mlp_w1, mlp_b1, mlp_w2,
           mlp_b2):
    B = champions.shape[0]
    bb = 128
    grid = B // bb

    T = jnp.zeros((XDIM, VEC_D), jnp.float32)
    T = T.at[0:60, 0:30].set(champ_table)
    for k in range(3):
        T = T.at[60 + 60 * k:120 + 60 * k, 30 + 10 * k:40 + 10 * k].set(item_table)
    for j in range(7):
        T = T.at[240 + 27 * j:267 + 27 * j, 60 + 8 * j:68 + 8 * j].set(trait_table)
    T = T.at[429:448, 116:135].set(jnp.eye(STATS_D, dtype=jnp.float32))

    b1 = mlp_b1.reshape(1, 27)
    b2 = mlp_b2.reshape(1, VEC_D)

    return pl.pallas_call(
        _emb_kernel,
        grid=(grid,),
        in_specs=[
            pl.BlockSpec((bb, N_CH, 30), lambda i: (i, 0, 0)),
            pl.BlockSpec((bb, 10), lambda i: (i, 0)),
            pl.BlockSpec((bb, 27), lambda i: (i, 0)),
            pl.BlockSpec((bb, 10), lambda i: (i, 0)),
            pl.BlockSpec((XDIM, VEC_D), lambda i: (0, 0)),
            pl.BlockSpec((60, VEC_D), lambda i: (0, 0)),
            pl.BlockSpec((8, VEC_D), lambda i: (0, 0)),
            pl.BlockSpec((27, 27), lambda i: (0, 0)),
            pl.BlockSpec((1, 27), lambda i: (0, 0)),
            pl.BlockSpec((27, VEC_D), lambda i: (0, 0)),
            pl.BlockSpec((1, VEC_D), lambda i: (0, 0)),
        ],
        out_specs=pl.BlockSpec((bb, 58, VEC_D), lambda i: (i, 0, 0)),
        out_shape=jax.ShapeDtypeStruct((B, 58, VEC_D), jnp.float32),
    )(champions, items, traits, scalars, T, bench_table, pid_table,
      mlp_w1, b1, mlp_w2, b2)


# speedup vs baseline: 10.8564x; 1.0394x over previous
import jax
import jax.numpy as jnp
from jax.experimental import pallas as pl

VEC_D = 135
N_CH = 37
N_BENCH = 10
N_SCAL = 6
N_PID = 4
STATS_D = 19
XDIM = 60 + 3 * 60 + 7 * 27 + STATS_D


def _onehot_f(col, n):
    idx = jnp.clip(jnp.floor(col), 0.0, float(n - 1)).astype(jnp.int32)
    i = jax.lax.broadcasted_iota(jnp.int32, (col.shape[0], n), 1)
    return (i == idx).astype(jnp.bfloat16)


def _emb_kernel(ch_ref, it_ref, tr_ref, sc_ref, t_ref, bench_ref, pid_ref,
                w1_ref, b1_ref, w2_ref, b2_ref, out_ref):
    bb = ch_ref.shape[0]

    ch3 = ch_ref[...]
    CH = jnp.concatenate([ch3[:, c, :] for c in range(N_CH)], axis=0)
    pieces = [_onehot_f(CH[:, 0:1], 60)]
    for k in range(3):
        pieces.append(_onehot_f(CH[:, 1 + k:2 + k], 60))
    for j in range(7):
        pieces.append(_onehot_f(CH[:, 4 + j:5 + j], 27))
    pieces.append(CH[:, 11:30].astype(jnp.bfloat16))
    X = jnp.concatenate(pieces, axis=1)
    rows = jax.lax.dot_general(X, t_ref[...], (((1,), (0,)), ((), ())),
                               preferred_element_type=jnp.float32)
    for c in range(N_CH):
        out_ref[:, c, :] = rows[c * bb:(c + 1) * bb, :]

    it = it_ref[...]
    IT = jnp.concatenate([it[:, k:k + 1] for k in range(N_BENCH)], axis=0)
    bench_rows = jax.lax.dot_general(_onehot_f(IT, 60), bench_ref[...],
                                     (((1,), (0,)), ((), ())),
                                     preferred_element_type=jnp.float32)
    for k in range(N_BENCH):
        out_ref[:, N_CH + k, :] = bench_rows[k * bb:(k + 1) * bb, :]


    h = jax.lax.dot_general(tr_ref[...], w1_ref[...], (((1,), (0,)), ((), ())),
                            preferred_element_type=jnp.float32) + b1_ref[...]
    h = jnp.maximum(h, 0.0)
    out_ref[:, 47, :] = jax.lax.dot_general(h, w2_ref[...],
                                            (((1,), (0,)), ((), ())),
                                            preferred_element_type=jnp.float32) + b2_ref[...]

    sc = sc_ref[...]
    step = 200.0 / (VEC_D - 1)
    rng = jax.lax.broadcasted_iota(jnp.int32, (bb, VEC_D), 1)
    for k in range(N_SCAL):
        v = jnp.clip(sc[:, 1 + k:2 + k], 0.0, 200.0)
        steps = v / step
        lower = jnp.floor(steps)
        um = steps - lower
        lower_i = lower.astype(jnp.int32)
        enc = (rng == lower_i).astype(jnp.float32) * (1.0 - um) + \
              (rng == lower_i + 1).astype(jnp.float32) * um
        out_ref[:, 48 + k, :] = enc

    PID = jnp.concatenate([sc[:, 7:8], sc[:, 8:9], sc[:, 9:10], sc[:, 0:1]],
                          axis=0)
    pid_rows = jax.lax.dot_general(_onehot_f(PID, 8), pid_ref[...],
                                   (((1,), (0,)), ((), ())),
                                   preferred_element_type=jnp.float32)
    for k in range(N_PID):
        out_ref[:, N_CH + N_BENCH + 1 + N_SCAL + k, :] = \
            pid_rows[k * bb:(k + 1) * bb, :]


def kernel(champions, items, traits, scalars, champ_table, item_table,
           trait_table, bench_table, pid_table, mlp_w1, mlp_b1, mlp_w2,
           mlp_b2):
    B = champions.shape[0]
    bb = 128
    grid = B // bb

    T = jnp.zeros((XDIM, VEC_D), jnp.float32)
    T = T.at[0:60, 0:30].set(champ_table)
    for k in range(3):
        T = T.at[60 + 60 * k:120 + 60 * k, 30 + 10 * k:40 + 10 * k].set(item_table)
    for j in range(7):
        T = T.at[240 + 27 * j:267 + 27 * j, 60 + 8 * j:68 + 8 * j].set(trait_table)
    T = T.at[429:448, 116:135].set(jnp.eye(STATS_D, dtype=jnp.float32))
    T = T.astype(jnp.bfloat16)

    b1 = mlp_b1.reshape(1, 27)
    b2 = mlp_b2.reshape(1, VEC_D)

    return pl.pallas_call(
        _emb_kernel,
        grid=(grid,),
        in_specs=[
            pl.BlockSpec((bb, N_CH, 30), lambda i: (i, 0, 0)),
            pl.BlockSpec((bb, 10), lambda i: (i, 0)),
            pl.BlockSpec((bb, 27), lambda i: (i, 0)),
            pl.BlockSpec((bb, 10), lambda i: (i, 0)),
            pl.BlockSpec((XDIM, VEC_D), lambda i: (0, 0)),
            pl.BlockSpec((60, VEC_D), lambda i: (0, 0)),
            pl.BlockSpec((8, VEC_D), lambda i: (0, 0)),
            pl.BlockSpec((27, 27), lambda i: (0, 0)),
            pl.BlockSpec((1, 27), lambda i: (0, 0)),
            pl.BlockSpec((27, VEC_D), lambda i: (0, 0)),
            pl.BlockSpec((1, VEC_D), lambda i: (0, 0)),
        ],
        out_specs=pl.BlockSpec((bb, 58, VEC_D), lambda i: (i, 0, 0)),
        out_shape=jax.ShapeDtypeStruct((B, 58, VEC_D), jnp.float32),
    )(champions, items, traits, scalars, T, bench_table.astype(jnp.bfloat16),
      pid_table.astype(jnp.bfloat16), mlp_w1, b1, mlp_w2, b2)


# speedup vs baseline: 18.8889x; 1.7399x over previous
import jax
import jax.numpy as jnp
from jax.experimental import pallas as pl

VEC_D = 135
N_CH = 37
N_BENCH = 10
N_SCAL = 6
N_PID = 4
STATS_D = 19
XDIM = 60 + 3 * 60 + 7 * 27 + STATS_D


def _onehot_f(col, n):
    idx = jnp.clip(jnp.floor(col), 0.0, float(n - 1)).astype(jnp.int32)
    i = jax.lax.broadcasted_iota(jnp.int32, (col.shape[0], n), 1)
    return (i == idx).astype(jnp.bfloat16)


def _emb_kernel(ch_ref, it_ref, tr_ref, sc_ref, t_ref, s_ref, pat_ref,
                lim_ref, bench_ref, pid_ref, w1_ref, b1_ref, w2_ref, b2_ref,
                out_ref):
    bb = ch_ref.shape[1]

    CH = ch_ref[...].reshape(N_CH * bb, 30)
    idsf = jnp.clip(jnp.floor(CH[:, 0:11]), 0.0, lim_ref[...])
    VAL = jax.lax.dot_general(idsf.astype(jnp.bfloat16), s_ref[...],
                              (((1,), (0,)), ((), ())),
                              preferred_element_type=jnp.float32)
    Xids = jnp.where(VAL.astype(jnp.bfloat16) == pat_ref[...],
                     jnp.bfloat16(1.0), jnp.bfloat16(0.0))
    X = jnp.concatenate([Xids, CH[:, 11:30].astype(jnp.bfloat16)], axis=1)
    rows = jax.lax.dot_general(X, t_ref[...], (((1,), (0,)), ((), ())),
                               preferred_element_type=jnp.float32)
    for c in range(N_CH):
        out_ref[:, c, :] = rows[c * bb:(c + 1) * bb, :]

    it = it_ref[...]
    IT = jnp.concatenate([it[:, k:k + 1] for k in range(N_BENCH)], axis=0)
    bench_rows = jax.lax.dot_general(_onehot_f(IT, 60), bench_ref[...],
                                     (((1,), (0,)), ((), ())),
                                     preferred_element_type=jnp.float32)
    for k in range(N_BENCH):
        out_ref[:, N_CH + k, :] = bench_rows[k * bb:(k + 1) * bb, :]


    h = jax.lax.dot_general(tr_ref[...], w1_ref[...], (((1,), (0,)), ((), ())),
                            preferred_element_type=jnp.float32) + b1_ref[...]
    h = jnp.maximum(h, 0.0)
    out_ref[:, 47, :] = jax.lax.dot_general(h, w2_ref[...],
                                            (((1,), (0,)), ((), ())),
                                            preferred_element_type=jnp.float32) + b2_ref[...]

    sc = sc_ref[...]
    step = 200.0 / (VEC_D - 1)
    rng = jax.lax.broadcasted_iota(jnp.int32, (bb, VEC_D), 1)
    for k in range(N_SCAL):
        v = jnp.clip(sc[:, 1 + k:2 + k], 0.0, 200.0)
        steps = v / step
        lower = jnp.floor(steps)
        um = steps - lower
        lower_i = lower.astype(jnp.int32)
        enc = (rng == lower_i).astype(jnp.float32) * (1.0 - um) + \
              (rng == lower_i + 1).astype(jnp.float32) * um
        out_ref[:, 48 + k, :] = enc

    PID = jnp.concatenate([sc[:, 7:8], sc[:, 8:9], sc[:, 9:10], sc[:, 0:1]],
                          axis=0)
    pidx = jnp.clip(jnp.floor(PID), 0.0, 7.0).astype(jnp.int32)
    pidx2 = jnp.broadcast_to(pidx, (pidx.shape[0], VEC_D))
    pid_rows = jnp.take_along_axis(pid_ref[...], pidx2, axis=0)
    for k in range(N_PID):
        out_ref[:, N_CH + N_BENCH + 1 + N_SCAL + k, :] = \
            pid_rows[k * bb:(k + 1) * bb, :]


def kernel(champions, items, traits, scalars, champ_table, item_table,
           trait_table, bench_table, pid_table, mlp_w1, mlp_b1, mlp_w2,
           mlp_b2):
    B = champions.shape[0]
    bb = 128
    grid = B // bb

    T = jnp.zeros((XDIM, VEC_D), jnp.float32)
    T = T.at[0:60, 0:30].set(champ_table)
    for k in range(3):
        T = T.at[60 + 60 * k:120 + 60 * k, 30 + 10 * k:40 + 10 * k].set(item_table)
    for j in range(7):
        T = T.at[240 + 27 * j:267 + 27 * j, 60 + 8 * j:68 + 8 * j].set(trait_table)
    T = T.at[429:448, 116:135].set(jnp.eye(STATS_D, dtype=jnp.float32))
    T = T.astype(jnp.bfloat16)

    seg_starts = [0, 60, 120, 180] + [240 + 27 * j for j in range(7)]
    seg_sizes = [60, 60, 60, 60] + [27] * 7
    S = jnp.zeros((11, XDIM - STATS_D), jnp.float32)
    pat_parts = []
    for p, (st, sz) in enumerate(zip(seg_starts, seg_sizes)):
        S = S.at[p, st:st + sz].set(1.0)
        pat_parts.append(jnp.arange(sz, dtype=jnp.float32))
    S = S.astype(jnp.bfloat16)
    PAT = jnp.concatenate(pat_parts).reshape(1, XDIM - STATS_D).astype(
        jnp.bfloat16)
    LIM = jnp.array([[59.0] * 4 + [26.0] * 7], jnp.float32)

    b1 = mlp_b1.reshape(1, 27)
    b2 = mlp_b2.reshape(1, VEC_D)

    return pl.pallas_call(
        _emb_kernel,
        grid=(grid,),
        in_specs=[
            pl.BlockSpec((N_CH, bb, 30), lambda i: (0, i, 0)),
            pl.BlockSpec((bb, 10), lambda i: (i, 0)),
            pl.BlockSpec((bb, 27), lambda i: (i, 0)),
            pl.BlockSpec((bb, 10), lambda i: (i, 0)),
            pl.BlockSpec((XDIM, VEC_D), lambda i: (0, 0)),
            pl.BlockSpec((11, XDIM - STATS_D), lambda i: (0, 0)),
            pl.BlockSpec((1, XDIM - STATS_D), lambda i: (0, 0)),
            pl.BlockSpec((1, 11), lambda i: (0, 0)),
            pl.BlockSpec((60, VEC_D), lambda i: (0, 0)),
            pl.BlockSpec((8, VEC_D), lambda i: (0, 0)),
            pl.BlockSpec((27, 27), lambda i: (0, 0)),
            pl.BlockSpec((1, 27), lambda i: (0, 0)),
            pl.BlockSpec((27, VEC_D), lambda i: (0, 0)),
            pl.BlockSpec((1, VEC_D), lambda i: (0, 0)),
        ],
        out_specs=pl.BlockSpec((bb, 58, VEC_D), lambda i: (i, 0, 0)),
        out_shape=jax.ShapeDtypeStruct((B, 58, VEC_D), jnp.float32),
    )(champions.transpose(1, 0, 2), items, traits, scalars, T, S, PAT, LIM,
      bench_table.astype(jnp.bfloat16), pid_table, mlp_w1, b1, mlp_w2, b2)
